# baseline (speedup 1.0000x reference)
import functools

import jax
import jax.numpy as jnp
from jax import lax
from jax.experimental import pallas as pl
from jax.experimental.pallas import tpu as pltpu
from jax.experimental.pallas import tpu_sc as plsc

N = 100000
S = 24
B = 4096
NC, NS = 2, 16
NW = NC * NS
BPW = B // NW
PACK = 8
RING = 32


@functools.lru_cache(maxsize=None)
def _build_season():
    mesh = plsc.VectorSubcoreMesh(core_axis_name="c", subcore_axis_name="s")

    @functools.partial(
        pl.kernel,
        mesh=mesh,
        out_type=jax.ShapeDtypeStruct((B, S), jnp.float32),
        scratch_types=[
            pltpu.VMEM((BPW,), jnp.int32),
            pltpu.VMEM((2, RING * PACK, S), jnp.float32),
            pltpu.VMEM((BPW, S), jnp.float32),
            pltpu.SemaphoreType.DMA,
            pltpu.SemaphoreType.DMA,
        ],
        compiler_params=pltpu.CompilerParams(
            use_tc_tiling_on_sc=True, needs_layout_passes=False),
    )
    def season_kernel(tab_hbm, idx_hbm, out_hbm, idx_v, ring, srows,
                      sem0, sem1):
        wid = lax.axis_index("s") * NC + lax.axis_index("c")
        base = wid * BPW
        pltpu.sync_copy(idx_hbm.at[pl.ds(base, BPW)], idx_v)
        lanes = lax.iota(jnp.int32, 16)
        n_rounds = BPW // RING
        sems = [sem0, sem1]

        def fire(rnd):
            j0 = rnd * RING
            buf = rnd % 2
            copies = []
            for g16 in range(RING // 16):
                vec = idx_v[pl.ds(j0 + g16 * 16, 16)]
                for l in range(16):
                    j = g16 * 16 + l
                    i = vec[l]
                    copies.append(pltpu.async_copy(
                        tab_hbm.at[i >> 3],
                        ring.at[buf, pl.ds(j * PACK, PACK), :], sems[buf]))
            return copies

        def extract(rnd, copies):
            for c in copies:
                c.wait()
            j0 = rnd * RING
            buf = rnd % 2
            for g in range(RING // 16):
                sl = pl.ds(j0 + g * 16, 16)
                sub = lax.rem(idx_v[sl], PACK)
                rowv = (g * 16 + lanes) * PACK + sub
                bufv = jnp.full((16,), buf, jnp.int32)
                posv = j0 + g * 16 + lanes
                for c in range(S):
                    cv = jnp.full((16,), c, jnp.int32)
                    vals = plsc.load_gather(ring, [bufv, rowv, cv])
                    plsc.store_scatter(srows, [posv, cv], vals)

        pending = fire(0)
        for rnd in range(n_rounds):
            nxt = fire(rnd + 1) if rnd + 1 < n_rounds else None
            extract(rnd, pending)
            pending = nxt
        pltpu.sync_copy(srows, out_hbm.at[pl.ds(base, BPW), :])

    return season_kernel


@functools.lru_cache(maxsize=None)
def _build_small():
    mesh = plsc.VectorSubcoreMesh(core_axis_name="c", subcore_axis_name="s")

    @functools.partial(
        pl.kernel,
        mesh=mesh,
        out_type=(
            jax.ShapeDtypeStruct((B,), jnp.float32),
            jax.ShapeDtypeStruct((B,), jnp.float32),
        ),
        scratch_types=[
            pltpu.VMEM((BPW,), jnp.int32),
            pltpu.VMEM((BPW,), jnp.int32),
            pltpu.VMEM((BPW, PACK), jnp.float32),
            pltpu.VMEM((BPW, PACK), jnp.float32),
            pltpu.VMEM((BPW,), jnp.float32),
            pltpu.VMEM((BPW,), jnp.float32),
            pltpu.SemaphoreType.DMA,
            pltpu.SemaphoreType.DMA,
        ],
        compiler_params=pltpu.CompilerParams(
            use_tc_tiling_on_sc=False, needs_layout_passes=False),
    )
    def small_kernel(lev_hbm, seas_hbm, idx_hbm, lev_out, seas_out,
                     idx_v, row_v, lev_rows, seas_rows, lev_v, seas_v,
                     sem0, sem1):
        wid = lax.axis_index("s") * NC + lax.axis_index("c")
        base = wid * BPW
        pltpu.sync_copy(idx_hbm.at[pl.ds(base, BPW)], idx_v)
        for j in range(BPW // 16):
            sl = pl.ds(j * 16, 16)
            row_v[sl] = lax.shift_right_logical(idx_v[sl], 3)
        c0 = pltpu.async_copy(lev_hbm.at[row_v], lev_rows, sem0)
        c1 = pltpu.async_copy(seas_hbm.at[row_v], seas_rows, sem1)
        c0.wait()
        c1.wait()
        pos0 = lax.iota(jnp.int32, 16)
        for j in range(BPW // 16):
            sl = pl.ds(j * 16, 16)
            lane = lax.rem(idx_v[sl], PACK)
            pos = pos0 + j * 16
            lev_v[sl] = plsc.load_gather(lev_rows, [pos, lane])
            seas_v[sl] = plsc.load_gather(seas_rows, [pos, lane])
        pltpu.sync_copy(lev_v, lev_out.at[pl.ds(base, BPW)])
        pltpu.sync_copy(seas_v, seas_out.at[pl.ds(base, BPW)])

    return small_kernel


def kernel(train, val, test, info_cat, idxs, add_nl_layer,
           init_lev_sms, init_seas_sms, init_seasonalities):
    idx32 = idxs.astype(jnp.int32)
    lev_flat, seas_flat = _build_small()(
        init_lev_sms.reshape(N // PACK, PACK),
        init_seas_sms.reshape(N // PACK, PACK), idx32)
    season = _build_season()(
        init_seasonalities.reshape(N // PACK, PACK, S), idx32)
    return (lev_flat.reshape(B, 1), seas_flat.reshape(B, 1), season)
